# baseline (speedup 1.0000x reference)
import math

import jax
import jax.numpy as jnp
from jax import lax
from jax.experimental import pallas as pl

_B = 4
_S = 8192
_D = 1024
_NSEG = 64
_SEGW = _S // _NSEG
_TOPK = 8
_NMACRO = 4
_PAD = 16
_NSRC = 13
_EPS = 1.1920928955078125e-07
_NEG = -3.0e38


def _reduce_body(h_ref, means_ref, lasts_ref):
    x = h_ref[...]
    means_ref[...] = jnp.mean(x, axis=2)
    lasts_ref[...] = x[:, :, _SEGW - 1, :]


def _dot(a, b):
    return lax.dot_general(a, b, (((1,), (0,)), ((), ())),
                           preferred_element_type=jnp.float32)


def _dot_t(a, b):
    return lax.dot_general(a, b, (((1,), (1,)), ((), ())),
                           preferred_element_type=jnp.float32)


def _ct(a, b):
    return lax.dot_general(a, b, (((0,), (0,)), ((), ())),
                           preferred_element_type=jnp.float32)


def _compile_body(means_ref, lasts_ref, nll_ref, q_ref,
                  ws_ref, wk_ref, wv_ref, wo_ref, out_ref):
    f32 = jnp.float32

    h_cols = []
    s_cols = []
    for b in range(_B):
        mb = means_ref[b]
        h_cols.append(jnp.sqrt(jnp.sum(mb * mb, axis=1, keepdims=True)))
        s_cols.append(jnp.mean(nll_ref[b], axis=1, keepdims=True))
    h = jnp.concatenate(h_cols, axis=1)
    s = jnp.concatenate(s_cols, axis=1)

    def _z(v):
        mu = jnp.mean(v, axis=0, keepdims=True)
        sd = jnp.sqrt(jnp.mean((v - mu) * (v - mu), axis=0, keepdims=True))
        return (v - mu) / jnp.maximum(sd, 1e-6)

    scores = _z(h) + _z(s)

    iota = lax.broadcasted_iota(jnp.int32, (_NSEG, _B), 0)
    active = jnp.ones((_NSEG, _B), dtype=jnp.bool_)
    for _ in range(_TOPK):
        sm = jnp.where(active, scores, _NEG)
        m = jnp.max(sm, axis=0, keepdims=True)
        cand = active & (sm >= m)
        ik = jnp.min(jnp.where(cand, iota, _NSEG), axis=0, keepdims=True)
        active = active & (iota != ik)
    sel = ~active
    sel32 = sel.astype(f32)

    tri = (lax.broadcasted_iota(jnp.int32, (_NSEG, _NSEG), 1)
           < lax.broadcasted_iota(jnp.int32, (_NSEG, _NSEG), 0)).astype(f32)
    rank = _dot(tri, sel32)

    piota = lax.broadcasted_iota(jnp.int32, (_NSEG, _PAD), 1)
    piota_f = piota.astype(f32)
    gi = lax.broadcasted_iota(jnp.int32, (_NSEG, _PAD), 0)
    macro_mean = jnp.where((piota >= _TOPK) & (piota < _TOPK + _NMACRO)
                           & ((gi // 16) == (piota - _TOPK)), 1.0 / 16.0, 0.0)
    macro_last = jnp.where((piota >= _TOPK) & (piota < _TOPK + _NMACRO)
                           & (gi == (piota - _TOPK) * 16 + 15), 1.0, 0.0)
    glob_mean = jnp.where(piota == _NSRC - 1, 1.0 / 64.0, 0.0)
    glob_last = jnp.where((piota == _NSRC - 1) & (gi == _NSEG - 1), 1.0, 0.0)

    left_parts = []
    right_parts = []
    for b in range(_B):
        sel_b = sel[:, b:b + 1]
        rank_b = rank[:, b:b + 1]
        onehot = jnp.where((rank_b == piota_f) & sel_b & (piota < _TOPK),
                           1.0, 0.0)
        m_mat = onehot + macro_mean + glob_mean
        l_mat = onehot + macro_last + glob_last
        left_parts.append(_ct(m_mat, means_ref[b]))
        right_parts.append(_ct(l_mat, lasts_ref[b]))
    left = jnp.concatenate(left_parts, axis=0)
    right = jnp.concatenate(right_parts, axis=0)

    ws = ws_ref[...]
    summ = _dot_t(left, ws[:, :_D]) + _dot_t(right, ws[:, _D:])
    ms = jnp.mean(summ * summ, axis=1, keepdims=True)
    sources = summ * lax.rsqrt(ms + _EPS)

    keys = _dot_t(sources, wk_ref[...])
    vals = _dot_t(sources, wv_ref[...])
    q = q_ref[...]
    att = _dot_t(q, keys) / math.sqrt(_D)
    cols = lax.broadcasted_iota(jnp.int32, (64, _PAD), 1)
    pad_mask = cols >= _NSRC
    prefix_parts = []
    for b in range(_B):
        a_b = jnp.where(pad_mask, _NEG, att[:, b * _PAD:(b + 1) * _PAD])
        a_b = a_b - jnp.max(a_b, axis=1, keepdims=True)
        e = jnp.exp(a_b)
        p_b = e / jnp.sum(e, axis=1, keepdims=True)
        prefix_parts.append(_dot(p_b, vals[b * _PAD:(b + 1) * _PAD]))
    prefix = jnp.concatenate(prefix_parts, axis=0)
    out = _dot_t(prefix, wo_ref[...])
    out_ref[...] = out.reshape(_B, 64, _D)


def kernel(prev_hidden, prev_nll, query, W_sum, W_k, W_v, W_o):
    f32 = jnp.float32
    h4 = prev_hidden.reshape(_B, _NSEG, _SEGW, _D)
    nll3 = prev_nll.reshape(_B, _NSEG, _SEGW)

    nb = 16
    means, lasts = pl.pallas_call(
        _reduce_body,
        grid=(_B, _NSEG // nb),
        in_specs=[pl.BlockSpec((1, nb, _SEGW, _D), lambda b, n: (b, n, 0, 0))],
        out_specs=[pl.BlockSpec((1, nb, _D), lambda b, n: (b, n, 0)),
                   pl.BlockSpec((1, nb, _D), lambda b, n: (b, n, 0))],
        out_shape=[jax.ShapeDtypeStruct((_B, _NSEG, _D), f32),
                   jax.ShapeDtypeStruct((_B, _NSEG, _D), f32)],
    )(h4)

    out = pl.pallas_call(
        _compile_body,
        in_specs=[
            pl.BlockSpec((_B, _NSEG, _D), lambda: (0, 0, 0)),
            pl.BlockSpec((_B, _NSEG, _D), lambda: (0, 0, 0)),
            pl.BlockSpec((_B, _NSEG, _SEGW), lambda: (0, 0, 0)),
            pl.BlockSpec((64, _D), lambda: (0, 0)),
            pl.BlockSpec((_D, 2 * _D), lambda: (0, 0)),
            pl.BlockSpec((_D, _D), lambda: (0, 0)),
            pl.BlockSpec((_D, _D), lambda: (0, 0)),
            pl.BlockSpec((_D, _D), lambda: (0, 0)),
        ],
        out_specs=pl.BlockSpec((_B, 64, _D), lambda: (0, 0, 0)),
        out_shape=jax.ShapeDtypeStruct((_B, 64, _D), f32),
    )(means, lasts, nll3, query, W_sum, W_k, W_v, W_o)
    return out
